# baseline (speedup 1.0000x reference)
import functools

import jax
import jax.numpy as jnp
from jax import lax
from jax.experimental import pallas as pl
from jax.experimental.pallas import tpu as pltpu
from jax.experimental.pallas import tpu_sc as plsc

B = 16384
L = 200
V = 512
D = 128
NLANG = 10
T = B * L

NC = 2
NS = 16
NW = NC * NS
TPW = T // NW
RPW = B // NW
CHUNK = L
NCHUNK = TPW // CHUNK
CPAD = 208
R = 3


def _table_body(emb_ref, lang_ref, wt_ref, wb_ref, b_ref, out_ref):
    ft = jnp.dot(emb_ref[...], wt_ref[...],
                 preferred_element_type=jnp.float32,
                 precision=lax.Precision.HIGHEST)
    lb = jnp.dot(lang_ref[...], wb_ref[...],
                 preferred_element_type=jnp.float32,
                 precision=lax.Precision.HIGHEST) + b_ref[...]
    out_ref[...] = ft[None, :, :] + lb[:, None, :]


def _combined_table(emb_table, lang_table, w_top, w_bot, bias_row):
    return pl.pallas_call(
        _table_body,
        out_shape=jax.ShapeDtypeStruct((NLANG, V, D), jnp.float32),
    )(emb_table, lang_table, w_top, w_bot, bias_row)


_GSPLIT = [(0, 104), (104, 96)]


def _sc_gather_body(tab_hbm, chars_hbm, lang_hbm, out_hbm,
                    chars_v0, chars_v1, chars_v2,
                    idx_v0, idx_v1, idx_v2,
                    rows_v0, rows_v1, rows_v2,
                    lang_v, tab_sp, isem, gsem, osem):
    chars_v = (chars_v0, chars_v1, chars_v2)
    idx_v = (idx_v0, idx_v1, idx_v2)
    rows_v = (rows_v0, rows_v1, rows_v2)
    c = lax.axis_index("c")
    s = lax.axis_index("s")
    wid = s * NC + c

    base = wid * TPW

    def chars_start(g, p):
        pltpu.make_async_copy(
            chars_hbm.at[pl.ds(base + g * CHUNK, CHUNK)],
            chars_v[p].at[pl.ds(0, CHUNK)], isem.at[p],
        ).start()

    def gather_start(g, p):
        pltpu.make_async_copy(
            chars_hbm.at[pl.ds(base, CHUNK)], chars_v[p].at[pl.ds(0, CHUNK)],
            isem.at[p],
        ).wait()

        @pl.when(g + 1 < NCHUNK)
        def _prefetch():
            chars_start(g + 1, (p + 1) % R)

        lwin = lang_v[pl.ds(g, 16)]
        off0 = lwin[0] * V
        for k in range(CPAD // 16):
            c16 = chars_v[p][pl.ds(k * 16, 16)]
            idx_v[p][pl.ds(k * 16, 16)] = c16 + off0

        for j0, jn in _GSPLIT:
            pltpu.make_async_copy(
                tab_sp.at[idx_v[p].at[pl.ds(j0, jn)]],
                rows_v[p].at[pl.ds(j0, jn)],
                gsem.at[p],
            ).start()

    def gather_wait(p):
        pltpu.make_async_copy(
            tab_hbm.at[pl.ds(0, CHUNK)], rows_v[p], gsem.at[p]
        ).wait()

    def out_start(g, p):
        pltpu.make_async_copy(
            rows_v[p], out_hbm.at[pl.ds(base + g * CHUNK, CHUNK)], osem.at[p]
        ).start()

    def out_wait(g, p):
        pltpu.make_async_copy(
            rows_v[p], out_hbm.at[pl.ds(base + g * CHUNK, CHUNK)], osem.at[p]
        ).wait()

    def step(g, pg):
        pn = (pg + 1) % R
        gather_wait(pg)
        out_start(g, pg)
        out_wait(g + 1 - R, pn)
        gather_start(g + 1, pn)

    chars_start(0, 0)
    pltpu.sync_copy(lang_hbm.at[pl.ds(wid * RPW, RPW)], lang_v.at[pl.ds(0, RPW)])
    srows = (NLANG * V) // NS
    pltpu.sync_copy(tab_hbm.at[pl.ds(s * srows, srows)],
                    tab_sp.at[pl.ds(s * srows, srows)])
    plsc.subcore_barrier()

    gather_start(0, 0)
    for q in range(1, R):
        gather_wait(q - 1)
        out_start(q - 1, q - 1)
        gather_start(q, q)

    nbody = (NCHUNK - R) // R
    def ring_body(i, carry):
        for q in range(R):
            step(R - 1 + i * R + q, (R - 1 + q) % R)
        return carry

    lax.fori_loop(0, nbody, ring_body, 0)

    for g in range(R - 1 + nbody * R, NCHUNK - 1):
        step(g, g % R)
    g_last = NCHUNK - 1
    p_last = g_last % R
    gather_wait(p_last)
    out_start(g_last, p_last)
    for q in range(R):
        out_wait(g_last - (R - 1) + q, (p_last + 1 + q) % R)


_sc_gather = functools.partial(
    pl.kernel,
    mesh=plsc.VectorSubcoreMesh(core_axis_name="c", subcore_axis_name="s"),
    out_type=jax.ShapeDtypeStruct((T, D), jnp.float32),
    scratch_types=(
        [pltpu.VMEM((CPAD,), jnp.int32)] * 3
        + [pltpu.VMEM((CPAD,), jnp.int32)] * 3
        + [pltpu.VMEM((CHUNK, D), jnp.float32)] * 3
        + [
            pltpu.VMEM((RPW + 16,), jnp.int32),
            pltpu.VMEM_SHARED((NLANG * V, D), jnp.float32),
            pltpu.SemaphoreType.DMA((R,)),
            pltpu.SemaphoreType.DMA((R,)),
            pltpu.SemaphoreType.DMA((R,)),
        ]
    ),
)(_sc_gather_body)


def kernel(chars, lang_id, emb_table, lang_table, proj_W, proj_b):
    w_top = proj_W[:D]
    w_bot = proj_W[D:]
    combined = _combined_table(emb_table, lang_table, w_top, w_bot,
                               proj_b.reshape(1, D))
    combined = combined.reshape(NLANG * V, D)
    out = _sc_gather(combined, chars.astype(jnp.int32).reshape(T),
                     lang_id.astype(jnp.int32))
    return out.reshape(B, L, D)


# speedup vs baseline: 1.0406x; 1.0406x over previous
import functools

import jax
import jax.numpy as jnp
from jax import lax
from jax.experimental import pallas as pl
from jax.experimental.pallas import tpu as pltpu
from jax.experimental.pallas import tpu_sc as plsc

B = 16384
L = 200
V = 512
D = 128
NLANG = 10
T = B * L

NC = 2
NS = 16
NW = NC * NS
TPW = T // NW
RPW = B // NW
CHUNK = L
NCHUNK = TPW // CHUNK
CPAD = 208
R = 3


def _table_body(emb_ref, lang_ref, wt_ref, wb_ref, b_ref, out_ref):
    ft = jnp.dot(emb_ref[...], wt_ref[...],
                 preferred_element_type=jnp.float32,
                 precision=lax.Precision.HIGHEST)
    lb = jnp.dot(lang_ref[...], wb_ref[...],
                 preferred_element_type=jnp.float32,
                 precision=lax.Precision.HIGHEST) + b_ref[...]
    out_ref[...] = ft[None, :, :] + lb[:, None, :]


def _combined_table(emb_table, lang_table, w_top, w_bot, bias_row):
    return pl.pallas_call(
        _table_body,
        out_shape=jax.ShapeDtypeStruct((NLANG, V, D), jnp.float32),
    )(emb_table, lang_table, w_top, w_bot, bias_row)


_GSPLIT = [(0, 104), (104, 96)]


def _sc_gather_body(tab_hbm, chars_hbm, lang_hbm, out_hbm,
                    chars_v0, chars_v1, chars_v2,
                    idx_v0, idx_v1, idx_v2,
                    rows_v0, rows_v1, rows_v2,
                    lang_v, tab_sp, isem, gsem, osem):
    chars_v = (chars_v0, chars_v1, chars_v2)
    idx_v = (idx_v0, idx_v1, idx_v2)
    rows_v = (rows_v0, rows_v1, rows_v2)
    c = lax.axis_index("c")
    s = lax.axis_index("s")
    wid = s * NC + c

    base = wid * TPW

    def chars_start(g, p):
        pltpu.make_async_copy(
            chars_hbm.at[pl.ds(base + g * CHUNK, CHUNK)],
            chars_v[p].at[pl.ds(0, CHUNK)], isem.at[p],
        ).start()

    def gather_start(g, p):
        pltpu.make_async_copy(
            chars_hbm.at[pl.ds(base, CHUNK)], chars_v[p].at[pl.ds(0, CHUNK)],
            isem.at[p],
        ).wait()

        @pl.when(g + 1 < NCHUNK)
        def _prefetch():
            chars_start(g + 1, (p + 1) % R)

        lwin = lang_v[pl.ds(g, 16)]
        off0 = lwin[0] * V
        for k in range(CPAD // 16):
            c16 = chars_v[p][pl.ds(k * 16, 16)]
            idx_v[p][pl.ds(k * 16, 16)] = c16 + off0

        for j0, jn in _GSPLIT:
            pltpu.make_async_copy(
                tab_sp.at[idx_v[p].at[pl.ds(j0, jn)]],
                rows_v[p].at[pl.ds(j0, jn)],
                gsem.at[p],
            ).start()

    def gather_wait(p):
        pltpu.make_async_copy(
            tab_hbm.at[pl.ds(0, CHUNK)], rows_v[p], gsem.at[p]
        ).wait()

    def out_start(g, p):
        pltpu.make_async_copy(
            rows_v[p], out_hbm.at[pl.ds(base + g * CHUNK, CHUNK)], osem.at[p]
        ).start()

    def out_wait(g, p):
        pltpu.make_async_copy(
            rows_v[p], out_hbm.at[pl.ds(base + g * CHUNK, CHUNK)], osem.at[p]
        ).wait()

    def step(g, pg):
        pn = (pg + 1) % R
        out_wait(g + 1 - R, pn)
        gather_start(g + 1, pn)
        gather_wait(pg)
        out_start(g, pg)

    chars_start(0, 0)
    pltpu.sync_copy(lang_hbm.at[pl.ds(wid * RPW, RPW)], lang_v.at[pl.ds(0, RPW)])
    srows = (NLANG * V) // NS
    pltpu.sync_copy(tab_hbm.at[pl.ds(s * srows, srows)],
                    tab_sp.at[pl.ds(s * srows, srows)])
    plsc.subcore_barrier()

    gather_start(0, 0)
    for q in range(1, R):
        gather_wait(q - 1)
        out_start(q - 1, q - 1)
        gather_start(q, q)

    nbody = (NCHUNK - R) // R
    def ring_body(i, carry):
        for q in range(R):
            step(R - 1 + i * R + q, (R - 1 + q) % R)
        return carry

    lax.fori_loop(0, nbody, ring_body, 0)

    for g in range(R - 1 + nbody * R, NCHUNK - 1):
        step(g, g % R)
    g_last = NCHUNK - 1
    p_last = g_last % R
    gather_wait(p_last)
    out_start(g_last, p_last)
    for q in range(R):
        out_wait(g_last - (R - 1) + q, (p_last + 1 + q) % R)


_sc_gather = functools.partial(
    pl.kernel,
    mesh=plsc.VectorSubcoreMesh(core_axis_name="c", subcore_axis_name="s"),
    out_type=jax.ShapeDtypeStruct((T, D), jnp.float32),
    scratch_types=(
        [pltpu.VMEM((CPAD,), jnp.int32)] * 3
        + [pltpu.VMEM((CPAD,), jnp.int32)] * 3
        + [pltpu.VMEM((CHUNK, D), jnp.float32)] * 3
        + [
            pltpu.VMEM((RPW + 16,), jnp.int32),
            pltpu.VMEM_SHARED((NLANG * V, D), jnp.float32),
            pltpu.SemaphoreType.DMA((R,)),
            pltpu.SemaphoreType.DMA((R,)),
            pltpu.SemaphoreType.DMA((R,)),
        ]
    ),
)(_sc_gather_body)


def kernel(chars, lang_id, emb_table, lang_table, proj_W, proj_b):
    w_top = proj_W[:D]
    w_bot = proj_W[D:]
    combined = _combined_table(emb_table, lang_table, w_top, w_bot,
                               proj_b.reshape(1, D))
    combined = combined.reshape(NLANG * V, D)
    out = _sc_gather(combined, chars.astype(jnp.int32).reshape(T),
                     lang_id.astype(jnp.int32))
    return out.reshape(B, L, D)
